# baseline (speedup 1.0000x reference)
import jax
import jax.numpy as jnp
from jax.experimental import pallas as pl
from jax.experimental.pallas import tpu as pltpu

_K = 2048
_D = 64
_R = 1024


def _vq_block_kernel(z_ref, w_ref, xq_ref, idx_ref, loss_ref):
    zb = z_ref[...]
    w = w_ref[...]

    zsq = jnp.sum(zb * zb, axis=1, keepdims=True)
    wsq = jnp.sum(w * w, axis=1)

    z2 = (zb.astype(jnp.bfloat16) * jnp.bfloat16(2.0))
    mm2 = jax.lax.dot_general(
        z2, w, (((1,), (1,)), ((), ())),
        preferred_element_type=jnp.float32,
    )

    dist = (zsq - mm2) + wsq[None, :]
    m = jnp.min(dist, axis=1, keepdims=True)
    kiota = jax.lax.broadcasted_iota(jnp.int32, dist.shape, 1)
    idx = jnp.min(jnp.where(dist == m, kiota, _K), axis=1)
    idx_ref[0, 0, :] = idx

    oh = (kiota == idx[:, None]).astype(jnp.bfloat16)
    w_hi = w.astype(jnp.bfloat16)
    r1 = w - w_hi.astype(jnp.float32)
    w_mid = r1.astype(jnp.bfloat16)
    w_lo = (r1 - w_mid.astype(jnp.float32)).astype(jnp.bfloat16)
    dn = (((1,), (0,)), ((), ()))
    xq = (
        jax.lax.dot_general(oh, w_hi, dn, preferred_element_type=jnp.float32)
        + jax.lax.dot_general(oh, w_mid, dn, preferred_element_type=jnp.float32)
        + jax.lax.dot_general(oh, w_lo, dn, preferred_element_type=jnp.float32)
    )

    xq_ref[...] = zb + (xq - zb)

    d = zb - xq
    psum = jnp.sum(d * d).reshape(1, 1)

    @pl.when(pl.program_id(0) == 0)
    def _init():
        loss_ref[...] = jnp.zeros((1, 1), jnp.float32)

    loss_ref[...] += psum


def kernel(z, codebook):
    B, T, D = z.shape
    N = B * T
    zf = z.reshape(N, D)
    nb = N // _R

    xq_st, idx, loss_sum = pl.pallas_call(
        _vq_block_kernel,
        grid=(nb,),
        in_specs=[
            pl.BlockSpec((_R, D), lambda i: (i, 0)),
            pl.BlockSpec((_K, D), lambda i: (0, 0)),
        ],
        out_specs=[
            pl.BlockSpec((_R, D), lambda i: (i, 0)),
            pl.BlockSpec((1, 1, _R), lambda i: (i, 0, 0)),
            pl.BlockSpec((1, 1), lambda i: (0, 0)),
        ],
        out_shape=[
            jax.ShapeDtypeStruct((N, D), jnp.float32),
            jax.ShapeDtypeStruct((nb, 1, _R), jnp.int32),
            jax.ShapeDtypeStruct((1, 1), jnp.float32),
        ],
        compiler_params=pltpu.CompilerParams(
            dimension_semantics=("arbitrary",),
        ),
    )(zf, codebook)

    commit_loss = loss_sum[0, 0] * jnp.float32(2.0 ** -21)
    return xq_st.reshape(B, T, D), idx.reshape(B, T), commit_loss


# speedup vs baseline: 1.1831x; 1.1831x over previous
import jax
import jax.numpy as jnp
from jax.experimental import pallas as pl
from jax.experimental.pallas import tpu as pltpu

_K = 2048
_D = 64
_R = 1024


def _vq_block_kernel(z_ref, w_ref, wt_ref, xq_ref, idx_ref, loss_ref):
    zb = z_ref[...]
    w = w_ref[...]
    wt = wt_ref[...]

    zsq = jnp.sum(zb * zb, axis=1, keepdims=True)
    wsq = jnp.sum(wt * wt, axis=0, keepdims=True)

    z2 = (zb.astype(jnp.bfloat16) * jnp.bfloat16(2.0))
    mm2 = jax.lax.dot_general(
        z2, wt, (((1,), (0,)), ((), ())),
        preferred_element_type=jnp.float32,
    )

    dist = (zsq - mm2) + wsq
    m = jnp.min(dist, axis=1, keepdims=True)
    kiota = jax.lax.broadcasted_iota(jnp.int32, dist.shape, 1)
    idx = jnp.min(jnp.where(dist == m, kiota, _K), axis=1)
    idx_ref[0, 0, :] = idx

    oh = (kiota == idx[:, None]).astype(jnp.bfloat16)
    w_hi = w.astype(jnp.bfloat16)
    w_lo = (w - w_hi.astype(jnp.float32)).astype(jnp.bfloat16)
    dn = (((1,), (0,)), ((), ()))
    xq = (
        jax.lax.dot_general(oh, w_hi, dn, preferred_element_type=jnp.float32)
        + jax.lax.dot_general(oh, w_lo, dn, preferred_element_type=jnp.float32)
    )

    xq_ref[...] = zb + (xq - zb)

    d = zb - xq
    loss_ref[...] = jnp.sum(d * d).reshape(1, 1, 1)


def kernel(z, codebook):
    B, T, D = z.shape
    N = B * T
    zf = z.reshape(N, D)
    nb = N // _R

    xq_st, idx, loss_parts = pl.pallas_call(
        _vq_block_kernel,
        grid=(nb,),
        in_specs=[
            pl.BlockSpec((_R, D), lambda i: (i, 0)),
            pl.BlockSpec((_K, D), lambda i: (0, 0)),
            pl.BlockSpec((D, _K), lambda i: (0, 0)),
        ],
        out_specs=[
            pl.BlockSpec((_R, D), lambda i: (i, 0)),
            pl.BlockSpec((1, 1, _R), lambda i: (i, 0, 0)),
            pl.BlockSpec((1, 1, 1), lambda i: (i, 0, 0)),
        ],
        out_shape=[
            jax.ShapeDtypeStruct((N, D), jnp.float32),
            jax.ShapeDtypeStruct((nb, 1, _R), jnp.int32),
            jax.ShapeDtypeStruct((nb, 1, 1), jnp.float32),
        ],
        compiler_params=pltpu.CompilerParams(
            dimension_semantics=("parallel",),
        ),
    )(zf, codebook, codebook.T)

    commit_loss = jnp.sum(loss_parts) * jnp.float32(2.0 ** -21)
    return xq_st.reshape(B, T, D), idx.reshape(B, T), commit_loss


# speedup vs baseline: 1.3890x; 1.1741x over previous
import jax
import jax.numpy as jnp
from jax.experimental import pallas as pl
from jax.experimental.pallas import tpu as pltpu

_K = 2048
_D = 64
_R = 1024


def _vq_block_kernel(zt_ref, w_ref, wt_ref, xqt_ref, idx_ref, loss_ref):
    ztb = zt_ref[0]
    w = w_ref[...]
    wt = wt_ref[...]

    zb = ztb.T

    zsq = jnp.sum(zb * zb, axis=1, keepdims=True)
    wsq = jnp.sum(wt * wt, axis=0, keepdims=True)

    z2 = (zb.astype(jnp.bfloat16) * jnp.bfloat16(2.0))
    mm2 = jax.lax.dot_general(
        z2, wt, (((1,), (0,)), ((), ())),
        preferred_element_type=jnp.float32,
    )

    dist = (zsq - mm2) + wsq
    idx = jnp.argmin(dist, axis=1).astype(jnp.int32)
    idx_ref[0, 0, :] = idx

    kiota = jax.lax.broadcasted_iota(jnp.int32, (_R, _K), 1)
    oh = (kiota == idx[:, None]).astype(jnp.bfloat16)
    w_hi = w.astype(jnp.bfloat16)
    w_lo = (w - w_hi.astype(jnp.float32)).astype(jnp.bfloat16)
    dn = (((1,), (0,)), ((), ()))
    xq = (
        jax.lax.dot_general(oh, w_hi, dn, preferred_element_type=jnp.float32)
        + jax.lax.dot_general(oh, w_lo, dn, preferred_element_type=jnp.float32)
    )

    xqt = xq.T
    xqt_ref[0] = ztb + (xqt - ztb)

    d = ztb - xqt
    loss_ref[...] = jnp.sum(d * d).reshape(1, 1, 1)


def kernel(z, codebook):
    B, T, D = z.shape
    N = B * T
    zt = jnp.transpose(z, (0, 2, 1))
    tiles = T // _R
    nb = N // _R

    xqt, idx, loss_parts = pl.pallas_call(
        _vq_block_kernel,
        grid=(nb,),
        in_specs=[
            pl.BlockSpec((1, D, _R), lambda i: (i // tiles, 0, i % tiles)),
            pl.BlockSpec((_K, D), lambda i: (0, 0)),
            pl.BlockSpec((D, _K), lambda i: (0, 0)),
        ],
        out_specs=[
            pl.BlockSpec((1, D, _R), lambda i: (i // tiles, 0, i % tiles)),
            pl.BlockSpec((1, 1, _R), lambda i: (i, 0, 0)),
            pl.BlockSpec((1, 1, 1), lambda i: (i, 0, 0)),
        ],
        out_shape=[
            jax.ShapeDtypeStruct((B, D, T), jnp.float32),
            jax.ShapeDtypeStruct((nb, 1, _R), jnp.int32),
            jax.ShapeDtypeStruct((nb, 1, 1), jnp.float32),
        ],
        compiler_params=pltpu.CompilerParams(
            dimension_semantics=("arbitrary",),
        ),
    )(zt, codebook, codebook.T)

    xq_st = jnp.transpose(xqt, (0, 2, 1))
    commit_loss = jnp.sum(loss_parts) * jnp.float32(2.0 ** -21)
    return xq_st, idx.reshape(B, T), commit_loss


# speedup vs baseline: 1.4172x; 1.0203x over previous
import jax
import jax.numpy as jnp
from jax.experimental import pallas as pl
from jax.experimental.pallas import tpu as pltpu

_K = 2048
_D = 64
_R = 2048


def _vq_block_kernel(zt_ref, w_ref, wt_ref, xqt_ref, idx_ref, loss_ref):
    ztb = zt_ref[0]
    w = w_ref[...]
    wt = wt_ref[...]

    zb = ztb.T

    zsq = jnp.sum(zb * zb, axis=1, keepdims=True)
    wsq = jnp.sum(wt * wt, axis=0, keepdims=True)

    z2 = (zb.astype(jnp.bfloat16) * jnp.bfloat16(2.0))
    mm2 = jax.lax.dot_general(
        z2, wt, (((1,), (0,)), ((), ())),
        preferred_element_type=jnp.float32,
    )

    dist = (zsq - mm2) + wsq
    idx = jnp.argmin(dist, axis=1).astype(jnp.int32)
    idx_ref[0, 0, :] = idx

    kiota = jax.lax.broadcasted_iota(jnp.int32, (_R, _K), 1)
    oh = (kiota == idx[:, None]).astype(jnp.bfloat16)
    w_hi = w.astype(jnp.bfloat16)
    w_lo = (w - w_hi.astype(jnp.float32)).astype(jnp.bfloat16)
    dn = (((1,), (0,)), ((), ()))
    xq = (
        jax.lax.dot_general(oh, w_hi, dn, preferred_element_type=jnp.float32)
        + jax.lax.dot_general(oh, w_lo, dn, preferred_element_type=jnp.float32)
    )

    xqt = xq.T
    xqt_ref[0] = ztb + (xqt - ztb)

    d = ztb - xqt
    loss_ref[...] = jnp.sum(d * d).reshape(1, 1, 1)


def kernel(z, codebook):
    B, T, D = z.shape
    N = B * T
    zt = jnp.transpose(z, (0, 2, 1))
    tiles = T // _R
    nb = N // _R

    xqt, idx, loss_parts = pl.pallas_call(
        _vq_block_kernel,
        grid=(nb,),
        in_specs=[
            pl.BlockSpec((1, D, _R), lambda i: (i // tiles, 0, i % tiles)),
            pl.BlockSpec((_K, D), lambda i: (0, 0)),
            pl.BlockSpec((D, _K), lambda i: (0, 0)),
        ],
        out_specs=[
            pl.BlockSpec((1, D, _R), lambda i: (i // tiles, 0, i % tiles)),
            pl.BlockSpec((1, 1, _R), lambda i: (i, 0, 0)),
            pl.BlockSpec((1, 1, 1), lambda i: (i, 0, 0)),
        ],
        out_shape=[
            jax.ShapeDtypeStruct((B, D, T), jnp.float32),
            jax.ShapeDtypeStruct((nb, 1, _R), jnp.int32),
            jax.ShapeDtypeStruct((nb, 1, 1), jnp.float32),
        ],
        compiler_params=pltpu.CompilerParams(
            dimension_semantics=("arbitrary",),
        ),
    )(zt, codebook, codebook.T)

    xq_st = jnp.transpose(xqt, (0, 2, 1))
    commit_loss = jnp.sum(loss_parts) * jnp.float32(2.0 ** -21)
    return xq_st, idx.reshape(B, T), commit_loss


# speedup vs baseline: 1.7477x; 1.2332x over previous
import jax
import jax.numpy as jnp
from jax.experimental import pallas as pl
from jax.experimental.pallas import tpu as pltpu

_K = 2048
_D = 64
_R = 2048


def _vq_block_kernel(zt_ref, wp_ref, wt_ref, xqt_ref, idx_ref, loss_ref):
    ztb = zt_ref[0]
    wp = wp_ref[...]
    wt = wt_ref[...]

    zb = ztb.T

    zsq = jnp.sum(zb * zb, axis=1, keepdims=True)
    wsq = jnp.sum(wt * wt, axis=0, keepdims=True)

    z2 = (zb.astype(jnp.bfloat16) * jnp.bfloat16(2.0))
    mm2 = jax.lax.dot_general(
        z2, wt, (((1,), (0,)), ((), ())),
        preferred_element_type=jnp.float32,
    )

    dist = (zsq - mm2) + wsq
    idx = jnp.argmin(dist, axis=1).astype(jnp.int32)
    idx_ref[0, 0, :] = idx

    kiota = jax.lax.broadcasted_iota(jnp.int32, (_R, _K), 1)
    oh = (kiota == idx[:, None]).astype(jnp.bfloat16)
    dn = (((1,), (0,)), ((), ()))
    xq_planes = jax.lax.dot_general(
        oh, wp, dn, preferred_element_type=jnp.float32)
    xq = xq_planes[:, :_D] + xq_planes[:, _D:]

    xqt = xq.T
    xqt_ref[0] = ztb + (xqt - ztb)

    d = ztb - xqt
    loss_ref[...] = jnp.sum(d * d).reshape(1, 1, 1)


def kernel(z, codebook):
    B, T, D = z.shape
    N = B * T
    zt = jnp.transpose(z, (0, 2, 1))
    tiles = T // _R
    nb = N // _R

    w_hi = codebook.astype(jnp.bfloat16)
    w_lo = (codebook - w_hi.astype(jnp.float32)).astype(jnp.bfloat16)
    w_planes = jnp.concatenate([w_hi, w_lo], axis=1)

    xqt, idx, loss_parts = pl.pallas_call(
        _vq_block_kernel,
        grid=(nb,),
        in_specs=[
            pl.BlockSpec((1, D, _R), lambda i: (i // tiles, 0, i % tiles)),
            pl.BlockSpec((_K, 2 * D), lambda i: (0, 0)),
            pl.BlockSpec((D, _K), lambda i: (0, 0)),
        ],
        out_specs=[
            pl.BlockSpec((1, D, _R), lambda i: (i // tiles, 0, i % tiles)),
            pl.BlockSpec((1, 1, _R), lambda i: (i, 0, 0)),
            pl.BlockSpec((1, 1, 1), lambda i: (i, 0, 0)),
        ],
        out_shape=[
            jax.ShapeDtypeStruct((B, D, T), jnp.float32),
            jax.ShapeDtypeStruct((nb, 1, _R), jnp.int32),
            jax.ShapeDtypeStruct((nb, 1, 1), jnp.float32),
        ],
        compiler_params=pltpu.CompilerParams(
            dimension_semantics=("arbitrary",),
        ),
    )(zt, w_planes, codebook.T)

    xq_st = jnp.transpose(xqt, (0, 2, 1))
    commit_loss = jnp.sum(loss_parts) * jnp.float32(2.0 ** -21)
    return xq_st, idx.reshape(B, T), commit_loss


# speedup vs baseline: 1.8739x; 1.0722x over previous
import functools

import jax
import jax.numpy as jnp
from jax.experimental import pallas as pl
from jax.experimental.pallas import tpu as pltpu
from jax.experimental.pallas import tpu_sc as plsc

_K = 2048
_D = 64
_R = 2048
_NW = 32


def _argmin_kernel(zt_ref, wt_ref, idx_ref):
    ztb = zt_ref[0]
    wt = wt_ref[...]

    zb = ztb.T

    zsq = jnp.sum(zb * zb, axis=1, keepdims=True)
    wsq = jnp.sum(wt * wt, axis=0, keepdims=True)

    z2 = (zb.astype(jnp.bfloat16) * jnp.bfloat16(2.0))
    mm2 = jax.lax.dot_general(
        z2, wt, (((1,), (0,)), ((), ())),
        preferred_element_type=jnp.float32,
    )

    dist = (zsq - mm2) + wsq
    idx = jnp.argmin(dist, axis=1).astype(jnp.int32)
    idx_ref[0, 0, :] = idx


def _sc_gather_kernel(table_hbm, idx_hbm, out_hbm, idx_v, rows_v, sem):
    bpw, half = idx_v.shape[0], rows_v.shape[0]
    wid = jax.lax.axis_index("s") * 2 + jax.lax.axis_index("c")
    base = wid * bpw
    pltpu.sync_copy(idx_hbm.at[pl.ds(base, bpw)], idx_v)
    for h in range(bpw // half):
        pltpu.async_copy(
            table_hbm.at[idx_v.at[pl.ds(h * half, half)]], rows_v, sem
        ).wait()
        pltpu.sync_copy(rows_v, out_hbm.at[pl.ds(base + h * half, half)])


def _st_loss_kernel(zt_ref, xq_ref, xqt_ref, loss_ref):
    ztb = zt_ref[0]
    xqt = xq_ref[:, :_D].T
    xqt_ref[0] = ztb + (xqt - ztb)
    d = ztb - xqt
    loss_ref[...] = jnp.sum(d * d).reshape(1, 1, 1)


def kernel(z, codebook):
    B, T, D = z.shape
    N = B * T
    zt = jnp.transpose(z, (0, 2, 1))
    tiles = T // _R
    nb = N // _R

    idx3 = pl.pallas_call(
        _argmin_kernel,
        grid=(nb,),
        in_specs=[
            pl.BlockSpec((1, D, _R), lambda i: (i // tiles, 0, i % tiles)),
            pl.BlockSpec((D, _K), lambda i: (0, 0)),
        ],
        out_specs=pl.BlockSpec((1, 1, _R), lambda i: (i, 0, 0)),
        out_shape=jax.ShapeDtypeStruct((nb, 1, _R), jnp.int32),
        compiler_params=pltpu.CompilerParams(
            dimension_semantics=("arbitrary",),
        ),
    )(zt, codebook.T)

    idx_flat = idx3.reshape(N)
    bpw = N // _NW
    wpad = jnp.pad(codebook, ((0, 0), (0, 128 - D)))
    mesh = plsc.VectorSubcoreMesh(core_axis_name="c", subcore_axis_name="s")
    xq = functools.partial(
        pl.kernel,
        mesh=mesh,
        out_type=jax.ShapeDtypeStruct((N, 128), jnp.float32),
        scratch_types=[
            pltpu.VMEM((bpw,), jnp.int32),
            pltpu.VMEM((bpw // 2, 128), jnp.float32),
            pltpu.SemaphoreType.DMA,
        ],
    )(_sc_gather_kernel)(wpad, idx_flat)

    xqt, loss_parts = pl.pallas_call(
        _st_loss_kernel,
        grid=(nb,),
        in_specs=[
            pl.BlockSpec((1, D, _R), lambda i: (i // tiles, 0, i % tiles)),
            pl.BlockSpec((_R, 128), lambda i: (i, 0)),
        ],
        out_specs=[
            pl.BlockSpec((1, D, _R), lambda i: (i // tiles, 0, i % tiles)),
            pl.BlockSpec((1, 1, 1), lambda i: (i, 0, 0)),
        ],
        out_shape=[
            jax.ShapeDtypeStruct((B, D, T), jnp.float32),
            jax.ShapeDtypeStruct((nb, 1, 1), jnp.float32),
        ],
        compiler_params=pltpu.CompilerParams(
            dimension_semantics=("arbitrary",),
        ),
    )(zt, xq)

    xq_st = jnp.transpose(xqt, (0, 2, 1))
    commit_loss = jnp.sum(loss_parts) * jnp.float32(2.0 ** -21)
    return xq_st, idx3.reshape(B, T), commit_loss
